# baseline (speedup 1.0000x reference)
import functools

import jax
import jax.numpy as jnp
from jax import lax
from jax.experimental import pallas as pl
from jax.experimental.pallas import tpu as pltpu
from jax.experimental.pallas import tpu_sc as plsc

_N, _C, _H, _W = 64, 256, 56, 56
_HW = _H * _W
_ROWS = _N * _C
_NC, _NS, _L = 2, 16, 16
_NW = _NC * _NS
_RPW = _ROWS // _NW
_G = 16
_NCHUNK = _RPW // _G
_CSETS = _C // _L


def _nz_body(mask_ref, sel_ref):
    m = mask_ref[0, :] != 0.0
    row = lax.broadcasted_iota(jnp.int32, (_C, _C), 0)
    col = lax.broadcasted_iota(jnp.int32, (_C, _C), 1)
    mcol = jnp.broadcast_to(m[None, :], (_C, _C))
    rank = jnp.sum(jnp.where(mcol & (col < row), 1, 0), axis=1)
    hit = mcol & (jnp.broadcast_to(rank[None, :], (_C, _C)) == row)
    sel_ref[0, :] = jnp.sum(jnp.where(hit, col, 0), axis=1)


_tc_nonzero = pl.pallas_call(
    _nz_body,
    out_shape=jax.ShapeDtypeStruct((1, _C), jnp.int32),
)


def _sc_body(x_hbm, sel_hbm, out_hbm, sel_v, idx_v, buf_v, sem0, sem1):
    wid = lax.axis_index("s") * _NC + lax.axis_index("c")
    pltpu.sync_copy(sel_hbm, sel_v)

    for g in range(_NCHUNK):
        n = 2 * wid + (g // _CSETS)
        off = jnp.full((_L,), n * _C, jnp.int32)
        idx_v[g, :] = sel_v[pl.ds((g % _CSETS) * _L, _L)] + off

    base = wid * _RPW
    sems = (sem0, sem1)
    desc = [None, None]
    desc[0] = pltpu.async_copy(x_hbm.at[idx_v.at[0]], buf_v.at[0], sems[0])
    for g in range(_NCHUNK):
        b = g & 1
        if g + 1 < _NCHUNK:
            desc[1 - b] = pltpu.async_copy(
                x_hbm.at[idx_v.at[g + 1]], buf_v.at[1 - b], sems[1 - b])
        desc[b].wait()
        pltpu.sync_copy(buf_v.at[b], out_hbm.at[pl.ds(base + g * _G, _G)])


_sc_gather = functools.partial(
    pl.kernel,
    out_type=jax.ShapeDtypeStruct((_ROWS, _HW), jnp.float32),
    mesh=plsc.VectorSubcoreMesh(core_axis_name="c", subcore_axis_name="s"),
    compiler_params=pltpu.CompilerParams(use_tc_tiling_on_sc=False),
    scratch_types=[
        pltpu.VMEM((_C,), jnp.int32),
        pltpu.VMEM((_NCHUNK, _L), jnp.int32),
        pltpu.VMEM((2, _G, _HW), jnp.float32),
        pltpu.SemaphoreType.DMA,
        pltpu.SemaphoreType.DMA,
    ],
)(_sc_body)


def kernel(input_tensor, indexes):
    sel = _tc_nonzero(indexes.reshape(1, _C)).reshape(_C)
    x2d = input_tensor.reshape(_ROWS, _HW)
    out2d = _sc_gather(x2d, sel)
    return out2d.reshape(_N, _C, _H, _W)
